# baseline (speedup 1.0000x reference)
import functools

import jax
import jax.numpy as jnp
from jax import lax
from jax.experimental import pallas as pl
from jax.experimental.pallas import tpu as pltpu
from jax.experimental.pallas import tpu_sc as plsc

DIM_IN = 64
NUM_INDS = 16
N_BINS = 64
B = 8
N = 65536

NC = 2
NS = 16
NW = NC * NS

N_SLICES = 2
B_SLICE = B // N_SLICES
ROWS_SLICE = B_SLICE * NUM_INDS
ROWS_PER_W = ROWS_SLICE // NW if ROWS_SLICE >= NW else 1
CHUNK_VALS = 32768
CHUNKS_PER_ROW = N // CHUNK_VALS
N_CHUNKS = ROWS_PER_W * CHUNKS_PER_ROW
HIST_W = 81
OUT_W = 64
CNT_SHIFT = 18
CNT_ONE = 1 << CNT_SHIFT
FRAC_Q = 64.0
UNROLL = 8
NBLK = 32768

assert ROWS_SLICE == NW * ROWS_PER_W


def _pos_body(iw_ref, x_ref, out_ref):
    s = lax.dot_general(iw_ref[...], x_ref[0],
                        (((1,), (0,)), ((), ())),
                        preferred_element_type=jnp.float32)
    v = jax.nn.sigmoid(s)
    out_ref[0] = jnp.clip(v * float(N_BINS) - 0.5, 0.0, float(N_BINS - 1))


def _compute_pos(Xt, Iw, s):
    grid = (B_SLICE, N // NBLK)
    return pl.pallas_call(
        _pos_body,
        grid=grid,
        in_specs=[
            pl.BlockSpec((NUM_INDS, DIM_IN), lambda b, j: (0, 0)),
            pl.BlockSpec((1, DIM_IN, NBLK),
                         lambda b, j: (b + s * B_SLICE, 0, j)),
        ],
        out_specs=pl.BlockSpec((1, NUM_INDS, NBLK), lambda b, j: (b, 0, j)),
        out_shape=jax.ShapeDtypeStruct((B_SLICE, NUM_INDS, N), jnp.float32),
    )(Iw, Xt)


def _hist_body(pos_hbm, out_hbm, buf0, buf1, hist, sv, stage, sem0, sem1):
    wid = lax.axis_index("s") * NC + lax.axis_index("c")
    base = wid * (ROWS_PER_W * N)

    zeros16 = jnp.zeros((16,), jnp.float32)
    zeros16i = jnp.zeros((16,), jnp.int32)
    lane_iota = lax.iota(jnp.int32, 16)
    lane_base = lane_iota * HIST_W
    inv_q = 1.0 / FRAC_Q

    bufs = [buf0, buf1]
    sems = [sem0, sem1]

    def _copy(c):
        return pltpu.make_async_copy(
            pos_hbm.at[pl.ds(base + c * CHUNK_VALS, CHUNK_VALS)],
            bufs[c % 2], sems[c % 2],
        )

    _copy(0).start()
    for c in range(N_CHUNKS):
        if c + 1 < N_CHUNKS:
            _copy(c + 1).start()
        if c % CHUNKS_PER_ROW == 0:
            for i in range(16 * HIST_W // 16):
                hist[pl.ds(i * 16, 16)] = zeros16i
        _copy(c).wait()
        buf = bufs[c % 2]

        @plsc.parallel_loop(0, CHUNK_VALS // 16, 1, unroll=UNROLL)
        def _vec(r):
            v = buf[pl.ds(r * 16, 16)]
            i0 = v.astype(jnp.int32)
            frac = v - i0.astype(jnp.float32)
            fq = jnp.minimum((frac * FRAC_Q + 0.5).astype(jnp.int32), 63)
            plsc.addupdate_scatter(hist, [lane_base + i0], fq + CNT_ONE)

        if c % CHUNKS_PER_ROW == CHUNKS_PER_ROW - 1:
            row = c // CHUNKS_PER_ROW
            sv[pl.ds(0, 16)] = zeros16
            for g in range(OUT_W // 16):
                cacc = zeros16
                sacc = zeros16
                for l in range(16):
                    a = plsc.load_gather(
                        hist, [lane_iota + jnp.int32(l * HIST_W + g * 16)])
                    cnt = jnp.right_shift(a, CNT_SHIFT)
                    s_i = jnp.bitwise_and(a, CNT_ONE - 1)
                    cacc = cacc + cnt.astype(jnp.float32)
                    sacc = sacc + s_i.astype(jnp.float32)
                plsc.store_scatter(sv, [lane_iota + jnp.int32(g * 16 + 1)],
                                   sacc)
                ssh = sv[pl.ds(g * 16, 16)]
                outv = cacc + (ssh - sacc) * inv_q
                if g == OUT_W // 16 - 1:
                    outv = jnp.where(lane_iota == 15, cacc + ssh * inv_q,
                                     outv)
                stage[pl.ds(row * OUT_W + g * 16, 16)] = outv

    pltpu.sync_copy(stage, out_hbm.at[pl.ds(wid * (ROWS_PER_W * OUT_W),
                                            ROWS_PER_W * OUT_W)])


_hist_call = functools.partial(
    pl.kernel,
    out_type=jax.ShapeDtypeStruct((ROWS_SLICE * OUT_W,), jnp.float32),
    mesh=plsc.VectorSubcoreMesh(core_axis_name="c", subcore_axis_name="s"),
    scratch_types=[
        pltpu.VMEM((CHUNK_VALS,), jnp.float32),
        pltpu.VMEM((CHUNK_VALS,), jnp.float32),
        pltpu.VMEM((16 * HIST_W,), jnp.int32),
        pltpu.VMEM((OUT_W + 16,), jnp.float32),
        pltpu.VMEM((ROWS_PER_W * OUT_W,), jnp.float32),
        pltpu.SemaphoreType.DMA,
        pltpu.SemaphoreType.DMA,
    ],
    compiler_params=pltpu.CompilerParams(needs_layout_passes=False),
)(_hist_body)


def kernel(X, I):
    Xt = X.transpose(0, 2, 1)
    Iw = I[0]
    parts = []
    for s in range(N_SLICES):
        pos = _compute_pos(Xt, Iw, s)
        parts.append(_hist_call(pos.reshape(ROWS_SLICE * N)))
    hist = jnp.concatenate(parts).reshape(B, NUM_INDS, OUT_W)
    hist = hist[:, :, :N_BINS] * (1.0 / N)
    return hist.reshape(B, NUM_INDS * N_BINS)


# speedup vs baseline: 1.0583x; 1.0583x over previous
import functools

import jax
import jax.numpy as jnp
from jax import lax
from jax.experimental import pallas as pl
from jax.experimental.pallas import tpu as pltpu
from jax.experimental.pallas import tpu_sc as plsc

DIM_IN = 64
NUM_INDS = 16
N_BINS = 64
B = 8
N = 65536

NC = 2
NS = 16
NW = NC * NS

N_SLICES = 1
B_SLICE = B // N_SLICES
ROWS_SLICE = B_SLICE * NUM_INDS
ROWS_PER_W = ROWS_SLICE // NW if ROWS_SLICE >= NW else 1
CHUNK_VALS = 32768
CHUNKS_PER_ROW = N // CHUNK_VALS
N_CHUNKS = ROWS_PER_W * CHUNKS_PER_ROW
HIST_W = 81
OUT_W = 64
CNT_SHIFT = 18
CNT_ONE = 1 << CNT_SHIFT
FRAC_Q = 64.0
UNROLL = 16
NBLK = 32768

assert ROWS_SLICE == NW * ROWS_PER_W


def _pos_body(iw_ref, x_ref, out_ref):
    s = lax.dot_general(iw_ref[...], x_ref[0],
                        (((1,), (0,)), ((), ())),
                        preferred_element_type=jnp.float32)
    v = jax.nn.sigmoid(s)
    out_ref[0] = jnp.clip(v * float(N_BINS) - 0.5, 0.0, float(N_BINS - 1))


def _compute_pos(Xt, Iw, s):
    grid = (B_SLICE, N // NBLK)
    return pl.pallas_call(
        _pos_body,
        grid=grid,
        in_specs=[
            pl.BlockSpec((NUM_INDS, DIM_IN), lambda b, j: (0, 0)),
            pl.BlockSpec((1, DIM_IN, NBLK),
                         lambda b, j: (b + s * B_SLICE, 0, j)),
        ],
        out_specs=pl.BlockSpec((1, NUM_INDS, NBLK), lambda b, j: (b, 0, j)),
        out_shape=jax.ShapeDtypeStruct((B_SLICE, NUM_INDS, N), jnp.float32),
    )(Iw, Xt)


def _hist_body(pos_hbm, out_hbm, buf0, buf1, hist, sv, stage, sem0, sem1):
    wid = lax.axis_index("s") * NC + lax.axis_index("c")
    base = wid * (ROWS_PER_W * N)

    zeros16 = jnp.zeros((16,), jnp.float32)
    zeros16i = jnp.zeros((16,), jnp.int32)
    lane_iota = lax.iota(jnp.int32, 16)
    lane_base = lane_iota * HIST_W
    inv_q = 1.0 / FRAC_Q

    bufs = [buf0, buf1]
    sems = [sem0, sem1]

    def _copy(c):
        return pltpu.make_async_copy(
            pos_hbm.at[pl.ds(base + c * CHUNK_VALS, CHUNK_VALS)],
            bufs[c % 2], sems[c % 2],
        )

    _copy(0).start()
    for c in range(N_CHUNKS):
        if c + 1 < N_CHUNKS:
            _copy(c + 1).start()
        if c % CHUNKS_PER_ROW == 0:
            for i in range(16 * HIST_W // 16):
                hist[pl.ds(i * 16, 16)] = zeros16i
        _copy(c).wait()
        buf = bufs[c % 2]

        @plsc.parallel_loop(0, CHUNK_VALS // 16, 1, unroll=UNROLL)
        def _vec(r):
            v = buf[pl.ds(r * 16, 16)]
            i0 = v.astype(jnp.int32)
            frac = v - i0.astype(jnp.float32)
            fq = jnp.minimum((frac * FRAC_Q + 0.5).astype(jnp.int32), 63)
            plsc.addupdate_scatter(hist, [lane_base + i0], fq + CNT_ONE)

        if c % CHUNKS_PER_ROW == CHUNKS_PER_ROW - 1:
            row = c // CHUNKS_PER_ROW
            sv[pl.ds(0, 16)] = zeros16
            for g in range(OUT_W // 16):
                cacc = zeros16
                sacc = zeros16
                for l in range(16):
                    a = plsc.load_gather(
                        hist, [lane_iota + jnp.int32(l * HIST_W + g * 16)])
                    cnt = jnp.right_shift(a, CNT_SHIFT)
                    s_i = jnp.bitwise_and(a, CNT_ONE - 1)
                    cacc = cacc + cnt.astype(jnp.float32)
                    sacc = sacc + s_i.astype(jnp.float32)
                plsc.store_scatter(sv, [lane_iota + jnp.int32(g * 16 + 1)],
                                   sacc)
                ssh = sv[pl.ds(g * 16, 16)]
                outv = cacc + (ssh - sacc) * inv_q
                if g == OUT_W // 16 - 1:
                    outv = jnp.where(lane_iota == 15, cacc + ssh * inv_q,
                                     outv)
                stage[pl.ds(row * OUT_W + g * 16, 16)] = outv

    pltpu.sync_copy(stage, out_hbm.at[pl.ds(wid * (ROWS_PER_W * OUT_W),
                                            ROWS_PER_W * OUT_W)])


_hist_call = functools.partial(
    pl.kernel,
    out_type=jax.ShapeDtypeStruct((ROWS_SLICE * OUT_W,), jnp.float32),
    mesh=plsc.VectorSubcoreMesh(core_axis_name="c", subcore_axis_name="s"),
    scratch_types=[
        pltpu.VMEM((CHUNK_VALS,), jnp.float32),
        pltpu.VMEM((CHUNK_VALS,), jnp.float32),
        pltpu.VMEM((16 * HIST_W,), jnp.int32),
        pltpu.VMEM((OUT_W + 16,), jnp.float32),
        pltpu.VMEM((ROWS_PER_W * OUT_W,), jnp.float32),
        pltpu.SemaphoreType.DMA,
        pltpu.SemaphoreType.DMA,
    ],
    compiler_params=pltpu.CompilerParams(needs_layout_passes=False),
)(_hist_body)


def kernel(X, I):
    Xt = X.transpose(0, 2, 1)
    Iw = I[0]
    parts = []
    for s in range(N_SLICES):
        pos = _compute_pos(Xt, Iw, s)
        parts.append(_hist_call(pos.reshape(ROWS_SLICE * N)))
    hist = jnp.concatenate(parts).reshape(B, NUM_INDS, OUT_W)
    hist = hist[:, :, :N_BINS] * (1.0 / N)
    return hist.reshape(B, NUM_INDS * N_BINS)


# speedup vs baseline: 1.0594x; 1.0011x over previous
import functools

import jax
import jax.numpy as jnp
from jax import lax
from jax.experimental import pallas as pl
from jax.experimental.pallas import tpu as pltpu
from jax.experimental.pallas import tpu_sc as plsc

DIM_IN = 64
NUM_INDS = 16
N_BINS = 64
B = 8
N = 65536

NC = 2
NS = 16
NW = NC * NS

N_SLICES = 1
B_SLICE = B // N_SLICES
ROWS_SLICE = B_SLICE * NUM_INDS
ROWS_PER_W = ROWS_SLICE // NW if ROWS_SLICE >= NW else 1
CHUNK_VALS = 32768
CHUNKS_PER_ROW = N // CHUNK_VALS
N_CHUNKS = ROWS_PER_W * CHUNKS_PER_ROW
HIST_W = 81
OUT_W = 64
CNT_SHIFT = 18
CNT_ONE = 1 << CNT_SHIFT
FRAC_Q = 64.0
UNROLL = 8
NBLK = 32768

assert ROWS_SLICE == NW * ROWS_PER_W


def _pos_body(iw_ref, x_ref, out_ref):
    s = lax.dot_general(iw_ref[...], x_ref[0],
                        (((1,), (0,)), ((), ())),
                        preferred_element_type=jnp.float32)
    v = jax.nn.sigmoid(s)
    out_ref[0] = jnp.clip(v * float(N_BINS) - 0.5, 0.0, float(N_BINS - 1))


def _compute_pos(Xt, Iw, s):
    grid = (B_SLICE, N // NBLK)
    return pl.pallas_call(
        _pos_body,
        grid=grid,
        in_specs=[
            pl.BlockSpec((NUM_INDS, DIM_IN), lambda b, j: (0, 0)),
            pl.BlockSpec((1, DIM_IN, NBLK),
                         lambda b, j: (b + s * B_SLICE, 0, j)),
        ],
        out_specs=pl.BlockSpec((1, NUM_INDS, NBLK), lambda b, j: (b, 0, j)),
        out_shape=jax.ShapeDtypeStruct((B_SLICE, NUM_INDS, N), jnp.float32),
    )(Iw, Xt)


def _hist_body(pos_hbm, out_hbm, buf0, buf1, hist, sv, stage, sem0, sem1):
    wid = lax.axis_index("s") * NC + lax.axis_index("c")
    base = wid * (ROWS_PER_W * N)

    zeros16 = jnp.zeros((16,), jnp.float32)
    zeros16i = jnp.zeros((16,), jnp.int32)
    lane_iota = lax.iota(jnp.int32, 16)
    lane_base = lane_iota * HIST_W
    inv_q = 1.0 / FRAC_Q

    bufs = [buf0, buf1]
    sems = [sem0, sem1]

    def _copy(c):
        return pltpu.make_async_copy(
            pos_hbm.at[pl.ds(base + c * CHUNK_VALS, CHUNK_VALS)],
            bufs[c % 2], sems[c % 2],
        )

    _copy(0).start()
    for c in range(N_CHUNKS):
        if c + 1 < N_CHUNKS:
            _copy(c + 1).start()
        if c % CHUNKS_PER_ROW == 0:
            for i in range(16 * HIST_W // 16):
                hist[pl.ds(i * 16, 16)] = zeros16i
        _copy(c).wait()
        buf = bufs[c % 2]

        @plsc.parallel_loop(0, CHUNK_VALS // 16, 1, unroll=UNROLL)
        def _vec(r):
            v = buf[pl.ds(r * 16, 16)]
            i0 = v.astype(jnp.int32)
            frac = v - i0.astype(jnp.float32)
            fq = jnp.minimum((frac * FRAC_Q + 0.5).astype(jnp.int32), 63)
            plsc.addupdate_scatter(hist, [lane_base + i0], fq + CNT_ONE)

        if c % CHUNKS_PER_ROW == CHUNKS_PER_ROW - 1:
            row = c // CHUNKS_PER_ROW
            sv[pl.ds(0, 16)] = zeros16
            for g in range(OUT_W // 16):
                cacc = zeros16
                sacc = zeros16
                for l in range(16):
                    a = plsc.load_gather(
                        hist, [lane_iota + jnp.int32(l * HIST_W + g * 16)])
                    cnt = jnp.right_shift(a, CNT_SHIFT)
                    s_i = jnp.bitwise_and(a, CNT_ONE - 1)
                    cacc = cacc + cnt.astype(jnp.float32)
                    sacc = sacc + s_i.astype(jnp.float32)
                plsc.store_scatter(sv, [lane_iota + jnp.int32(g * 16 + 1)],
                                   sacc)
                ssh = sv[pl.ds(g * 16, 16)]
                outv = cacc + (ssh - sacc) * inv_q
                if g == OUT_W // 16 - 1:
                    outv = jnp.where(lane_iota == 15, cacc + ssh * inv_q,
                                     outv)
                stage[pl.ds(row * OUT_W + g * 16, 16)] = outv

    pltpu.sync_copy(stage, out_hbm.at[pl.ds(wid * (ROWS_PER_W * OUT_W),
                                            ROWS_PER_W * OUT_W)])


_hist_call = functools.partial(
    pl.kernel,
    out_type=jax.ShapeDtypeStruct((ROWS_SLICE * OUT_W,), jnp.float32),
    mesh=plsc.VectorSubcoreMesh(core_axis_name="c", subcore_axis_name="s"),
    scratch_types=[
        pltpu.VMEM((CHUNK_VALS,), jnp.float32),
        pltpu.VMEM((CHUNK_VALS,), jnp.float32),
        pltpu.VMEM((16 * HIST_W,), jnp.int32),
        pltpu.VMEM((OUT_W + 16,), jnp.float32),
        pltpu.VMEM((ROWS_PER_W * OUT_W,), jnp.float32),
        pltpu.SemaphoreType.DMA,
        pltpu.SemaphoreType.DMA,
    ],
    compiler_params=pltpu.CompilerParams(needs_layout_passes=False),
)(_hist_body)


def kernel(X, I):
    Xt = X.transpose(0, 2, 1)
    Iw = I[0]
    pos = _compute_pos(Xt, Iw, 0)
    hist = _hist_call(pos.reshape(ROWS_SLICE * N))
    return hist.reshape(B, NUM_INDS * N_BINS) * (1.0 / N)
